# Initial kernel scaffold; baseline (speedup 1.0000x reference)
import jax
import jax.numpy as jnp
from jax.experimental import pallas as pl


def kernel(x, edge_index, edge_weight, W1, b1, W2, b2):
    raise NotImplementedError("write your pallas kernel here")


# speedup vs baseline: 11.7445x; 11.7445x over previous
import dataclasses
import functools

import jax
import jax.numpy as jnp
from jax import lax
from jax.experimental import pallas as pl
from jax.experimental.pallas import tpu as pltpu
from jax.experimental.pallas import tpu_sc as plsc

N = 10000
D = 256
E = 160000

NP = 10240
EP = 163840
NC = 2
NT = 16
LN = 16
K = 128
CH1 = EP // NT // K
CH2 = EP // (NC * NT) // K
RPT = NP // NT

_mesh = plsc.VectorSubcoreMesh(core_axis_name="c", subcore_axis_name="s")

_sc_params = pltpu.CompilerParams(needs_layout_passes=False,
                                  use_tc_tiling_on_sc=False)

_f32 = jnp.float32
_i32 = jnp.int32


def _deg_body(dst_hbm, w_hbm, out_hbm, dst_v, w_v, zb_v, acc_sh, sem):
    c = lax.axis_index("c")
    s = lax.axis_index("s")
    wid = c * NT + s

    @pl.loop(0, RPT // LN)
    def _(i):
        zb_v[pl.ds(i * LN, LN)] = jnp.zeros((LN,), _f32)

    pltpu.sync_copy(zb_v, acc_sh.at[pl.ds(s * RPT, RPT)])
    plsc.subcore_barrier()

    pltpu.sync_copy(dst_hbm.at[wid], dst_v)
    pltpu.sync_copy(w_hbm.at[wid], w_v)

    descs = []
    for j in range(CH2):
        descs.append(
            pltpu.async_copy(w_v.at[j], acc_sh.at[dst_v.at[j]], sem, add=True))
    for d in descs:
        d.wait()
    plsc.subcore_barrier()
    pltpu.sync_copy(acc_sh.at[pl.ds(s * RPT, RPT)],
                    out_hbm.at[c].at[pl.ds(s * RPT, RPT)])


_deg_call = pl.kernel(
    _deg_body,
    out_type=jax.ShapeDtypeStruct((NC, NP), _f32),
    mesh=_mesh,
    scratch_types=[
        pltpu.VMEM((CH2, K), _i32),
        pltpu.VMEM((CH2, K), _f32),
        pltpu.VMEM((RPT,), _f32),
        pltpu.VMEM_SHARED((NP,), _f32),
        pltpu.SemaphoreType.DMA,
    ],
    compiler_params=_sc_params,
)


_NS = 2
_NR = CH1 // _NS


def _agg1_body(g4_hbm, src_hbm, dst_hbm, w_hbm, out_hbm,
               src_v, dst_v, w_v,
               gb0, gb1, sb0, sb1, acc_sh,
               gm0, gm1, sm0, sm1):
    c = lax.axis_index("c")
    s = lax.axis_index("s")
    gbufs = (gb0, gb1)
    sbufs = (sb0, sb1)
    gsems = (gm0, gm1)
    ssems = (sm0, sm1)

    pltpu.sync_copy(dst_hbm.at[s], dst_v)
    pltpu.sync_copy(w_hbm.at[s], w_v)

    def issue_gather(j, slot):
        pltpu.async_copy(g4_hbm.at[src_v.at[pl.ds(j * K, K)]],
                         gbufs[slot], gsems[slot])

    def wait_gather(slot):
        pltpu.make_async_copy(g4_hbm.at[src_v.at[pl.ds(0, K)]],
                              gbufs[slot], gsems[slot]).wait()

    def issue_scatter(j, slot):
        pltpu.async_copy(sbufs[slot], acc_sh.at[dst_v.at[j]], ssems[slot],
                         add=True)

    def wait_scatter(slot):
        pltpu.make_async_copy(sbufs[slot], acc_sh.at[dst_v.at[0]],
                              ssems[slot]).wait()

    for p in range(2):
        @pl.loop(0, K)
        def _(r):
            for k in range(4):
                sb0[r, pl.ds(k * LN, LN)] = jnp.zeros((LN,), _f32)

        for qq in range(RPT // K):
            pltpu.sync_copy(sb0, acc_sh.at[pl.ds(s * RPT + qq * K, K)])

        pltpu.sync_copy(src_hbm.at[s], src_v)
        offv = jnp.full((LN,), c * (2 * NP) + p, _i32)

        @pl.loop(0, (CH1 * K) // LN)
        def _(i):
            v = src_v[pl.ds(i * LN, LN)]
            src_v[pl.ds(i * LN, LN)] = v + v + offv

        plsc.subcore_barrier()

        for slot in range(_NS):
            issue_gather(slot, slot)

        @pl.loop(0, _NR)
        def _(r):
            for slot in range(_NS):
                j = r * _NS + slot
                wait_gather(slot)

                @pl.when(r > 0)
                def _():
                    wait_scatter(slot)

                gb = gbufs[slot]
                sb = sbufs[slot]
                jb = j * K

                @pl.loop(0, K)
                def _(e):
                    widx = jnp.full((LN,), jb + e, _i32)
                    wv = plsc.load_gather(w_v, [widx])
                    for k in range(4):
                        sb[e, pl.ds(k * LN, LN)] = gb[e, pl.ds(k * LN, LN)] * wv

                issue_scatter(j, slot)

                @pl.when(r < _NR - 1)
                def _():
                    issue_gather(j + _NS, slot)

        for slot in range(_NS):
            wait_scatter(slot)
        plsc.subcore_barrier()
        q = c * 2 + p
        pltpu.sync_copy(acc_sh.at[pl.ds(s * RPT, RPT)],
                        out_hbm.at[q].at[pl.ds(s * RPT, RPT)])


_agg1_call = pl.kernel(
    _agg1_body,
    out_type=jax.ShapeDtypeStruct((2 * NC, NP, 64), _f32),
    mesh=_mesh,
    scratch_types=[
        pltpu.VMEM((CH1 * K,), _i32),
        pltpu.VMEM((CH1, K), _i32),
        pltpu.VMEM((CH1 * K,), _f32),
        pltpu.VMEM((K, 64), _f32),
        pltpu.VMEM((K, 64), _f32),
        pltpu.VMEM((K, 64), _f32),
        pltpu.VMEM((K, 64), _f32),
        pltpu.VMEM_SHARED((NP, 64), _f32),
        pltpu.SemaphoreType.DMA,
        pltpu.SemaphoreType.DMA,
        pltpu.SemaphoreType.DMA,
        pltpu.SemaphoreType.DMA,
    ],
    compiler_params=_sc_params,
)


def _agg2_body(zg_hbm, src_hbm, dst_hbm, w_hbm, out_hbm,
               zg_v, src_v, dst_v, w_v, vals_v, zb_v, acc_sh, sem):
    c = lax.axis_index("c")
    s = lax.axis_index("s")
    wid = c * NT + s

    @pl.loop(0, RPT // LN)
    def _(i):
        zb_v[pl.ds(i * LN, LN)] = jnp.zeros((LN,), _f32)

    pltpu.sync_copy(zb_v, acc_sh.at[pl.ds(s * RPT, RPT)])

    pltpu.sync_copy(zg_hbm, zg_v)
    pltpu.sync_copy(src_hbm.at[wid], src_v)
    pltpu.sync_copy(dst_hbm.at[wid], dst_v)
    pltpu.sync_copy(w_hbm.at[wid], w_v)
    plsc.subcore_barrier()

    @pl.loop(0, CH2 * K // LN)
    def _(i):
        idx = src_v[pl.ds(i * LN, LN)]
        vals = plsc.load_gather(zg_v, [idx])
        vals_v[pl.ds(i * LN, LN)] = vals * w_v[pl.ds(i * LN, LN)]

    descs = []
    for j in range(CH2):
        descs.append(
            pltpu.async_copy(vals_v.at[pl.ds(j * K, K)],
                             acc_sh.at[dst_v.at[j]], sem, add=True))
    for d in descs:
        d.wait()
    plsc.subcore_barrier()
    pltpu.sync_copy(acc_sh.at[pl.ds(s * RPT, RPT)],
                    out_hbm.at[c].at[pl.ds(s * RPT, RPT)])


_agg2_call = pl.kernel(
    _agg2_body,
    out_type=jax.ShapeDtypeStruct((NC, NP), _f32),
    mesh=_mesh,
    scratch_types=[
        pltpu.VMEM((NP,), _f32),
        pltpu.VMEM((CH2 * K,), _i32),
        pltpu.VMEM((CH2, K), _i32),
        pltpu.VMEM((CH2 * K,), _f32),
        pltpu.VMEM((CH2 * K,), _f32),
        pltpu.VMEM((RPT,), _f32),
        pltpu.VMEM_SHARED((NP,), _f32),
        pltpu.SemaphoreType.DMA,
    ],
    compiler_params=_sc_params,
)


def _mm_body(x_ref, w_ref, o_ref):
    o_ref[...] = jnp.dot(x_ref[...], w_ref[...],
                         preferred_element_type=_f32)


def _b_body(degp_ref, h_ref, g2_ref, dis_ref):
    dp = degp_ref[...]
    dis = lax.rsqrt(dp[:, 0:1] + dp[:, 1:2] + 1.0)
    g2_ref[...] = h_ref[...] * dis
    dis_ref[...] = dis


def _c_body(a0_ref, a1_ref, a2_ref, a3_ref, glo_ref, ghi_ref, dis_ref,
            w2_ref, b1_ref, zg_ref):
    dis = dis_ref[...]
    alo = jnp.concatenate([a0_ref[0], a1_ref[0]], axis=1)
    ahi = jnp.concatenate([a2_ref[0], a3_ref[0]], axis=1)
    rlo = jnp.maximum(dis * (alo + glo_ref[...]) + b1_ref[:, 0:128],
                      0.0)
    rhi = jnp.maximum(dis * (ahi + ghi_ref[...]) + b1_ref[:, 128:256],
                      0.0)
    z = (jnp.dot(rlo, w2_ref[0:128, :], preferred_element_type=_f32)
         + jnp.dot(rhi, w2_ref[128:256, :], preferred_element_type=_f32))
    zg_ref[...] = dis * z


def _d_body(a2_ref, zg_ref, dis_ref, b2_ref, o_ref):
    a2 = a2_ref[0] + a2_ref[1]
    o_ref[...] = dis_ref[...] * (a2 + zg_ref[...]) + b2_ref[0, 0]


_BS = 1024
_GR = NP // _BS


def kernel(x, edge_index, edge_weight, W1, b1, W2, b2):
    src = edge_index[0]
    dst = edge_index[1]
    pad = EP - E
    src_p = jnp.concatenate([src, jnp.zeros((pad,), _i32)])
    dst_p = jnp.concatenate([dst, jnp.zeros((pad,), _i32)])
    w_p = jnp.concatenate([edge_weight, jnp.zeros((pad,), _f32)])
    x_p = jnp.pad(x, ((0, NP - N), (0, 0)))

    src_a = src_p.reshape(NT, CH1 * K)
    dst_a = dst_p.reshape(NT, CH1, K)
    w_a = w_p.reshape(NT, CH1 * K)
    src_d = src_p.reshape(NC * NT, CH2 * K)
    dst_d = dst_p.reshape(NC * NT, CH2, K)
    w_d = w_p.reshape(NC * NT, CH2, K)
    w_d_flat = w_p.reshape(NC * NT, CH2 * K)

    deg_p = _deg_call(dst_d, w_d)

    h = pl.pallas_call(
        _mm_body,
        grid=(_GR,),
        in_specs=[pl.BlockSpec((_BS, D), lambda i: (i, 0)),
                  pl.BlockSpec((D, D), lambda i: (0, 0))],
        out_specs=pl.BlockSpec((_BS, D), lambda i: (i, 0)),
        out_shape=jax.ShapeDtypeStruct((NP, D), _f32),
    )(x_p, W1)

    degp_t = deg_p.T
    g2, dis = pl.pallas_call(
        _b_body,
        grid=(_GR, 2),
        in_specs=[pl.BlockSpec((_BS, NC), lambda i, c: (i, 0)),
                  pl.BlockSpec((_BS, 128), lambda i, c: (i, c))],
        out_specs=[pl.BlockSpec((_BS, 128), lambda i, c: (c * _GR + i, 0)),
                   pl.BlockSpec((_BS, 1), lambda i, c: (i, 0))],
        out_shape=[jax.ShapeDtypeStruct((NC * NP, 128), _f32),
                   jax.ShapeDtypeStruct((NP, 1), _f32)],
    )(degp_t, h)

    acc = _agg1_call(g2.reshape(4 * NP, 64), src_a, dst_a, w_a)

    zg = pl.pallas_call(
        _c_body,
        grid=(_GR,),
        in_specs=[pl.BlockSpec((1, _BS, 64), lambda i: (0, i, 0)),
                  pl.BlockSpec((1, _BS, 64), lambda i: (1, i, 0)),
                  pl.BlockSpec((1, _BS, 64), lambda i: (2, i, 0)),
                  pl.BlockSpec((1, _BS, 64), lambda i: (3, i, 0)),
                  pl.BlockSpec((_BS, 128), lambda i: (i, 0)),
                  pl.BlockSpec((_BS, 128), lambda i: (_GR + i, 0)),
                  pl.BlockSpec((_BS, 1), lambda i: (i, 0)),
                  pl.BlockSpec((D, 1), lambda i: (0, 0)),
                  pl.BlockSpec((1, D), lambda i: (0, 0))],
        out_specs=pl.BlockSpec((_BS, 1), lambda i: (i, 0)),
        out_shape=jax.ShapeDtypeStruct((NP, 1), _f32),
    )(acc, acc, acc, acc, g2, g2, dis, W2, b1.reshape(1, D))

    acc2 = _agg2_call(zg.reshape(NP), src_d, dst_d, w_d_flat)

    out = pl.pallas_call(
        _d_body,
        out_shape=jax.ShapeDtypeStruct((NP // 128, 128), _f32),
    )(acc2.reshape(NC, NP // 128, 128), zg.reshape(NP // 128, 128),
      dis.reshape(NP // 128, 128), b2.reshape(1, 1))

    return out.reshape(NP)[:N]


# speedup vs baseline: 12.3278x; 1.0497x over previous
import dataclasses
import functools

import jax
import jax.numpy as jnp
from jax import lax
from jax.experimental import pallas as pl
from jax.experimental.pallas import tpu as pltpu
from jax.experimental.pallas import tpu_sc as plsc

N = 10000
D = 256
E = 160000

NP = 10240
EP = 163840
NC = 2
NT = 16
LN = 16
K = 128
CH1 = EP // NT // K
CH2 = EP // (NC * NT) // K
RPT = NP // NT

_mesh = plsc.VectorSubcoreMesh(core_axis_name="c", subcore_axis_name="s")

_sc_params = pltpu.CompilerParams(needs_layout_passes=False,
                                  use_tc_tiling_on_sc=False)

_f32 = jnp.float32
_i32 = jnp.int32


def _deg_body(dst_hbm, w_hbm, out_hbm, dst_v, w_v, zb_v, acc_sh, sem):
    c = lax.axis_index("c")
    s = lax.axis_index("s")
    wid = c * NT + s

    @pl.loop(0, RPT // LN)
    def _(i):
        zb_v[pl.ds(i * LN, LN)] = jnp.zeros((LN,), _f32)

    pltpu.sync_copy(zb_v, acc_sh.at[pl.ds(s * RPT, RPT)])
    plsc.subcore_barrier()

    pltpu.sync_copy(dst_hbm.at[wid], dst_v)
    pltpu.sync_copy(w_hbm.at[wid], w_v)

    descs = []
    for j in range(CH2):
        descs.append(
            pltpu.async_copy(w_v.at[j], acc_sh.at[dst_v.at[j]], sem, add=True))
    for d in descs:
        d.wait()
    plsc.subcore_barrier()
    pltpu.sync_copy(acc_sh.at[pl.ds(s * RPT, RPT)],
                    out_hbm.at[c].at[pl.ds(s * RPT, RPT)])


_deg_call = pl.kernel(
    _deg_body,
    out_type=jax.ShapeDtypeStruct((NC, NP), _f32),
    mesh=_mesh,
    scratch_types=[
        pltpu.VMEM((CH2, K), _i32),
        pltpu.VMEM((CH2, K), _f32),
        pltpu.VMEM((RPT,), _f32),
        pltpu.VMEM_SHARED((NP,), _f32),
        pltpu.SemaphoreType.DMA,
    ],
    compiler_params=_sc_params,
)


_NS = 2
_NR = CH1 // _NS


def _agg1_body(g4_hbm, src_hbm, dst_hbm, w_hbm, out_hbm,
               src_v, dst_v, w_v,
               gb0, gb1, sb0, sb1, acc_sh,
               gm0, gm1, sm0, sm1):
    c = lax.axis_index("c")
    s = lax.axis_index("s")
    gbufs = (gb0, gb1)
    sbufs = (sb0, sb1)
    gsems = (gm0, gm1)
    ssems = (sm0, sm1)

    pltpu.sync_copy(dst_hbm.at[s], dst_v)
    pltpu.sync_copy(w_hbm.at[s], w_v)

    def issue_gather(j, slot):
        pltpu.async_copy(g4_hbm.at[src_v.at[pl.ds(j * K, K)]],
                         gbufs[slot], gsems[slot])

    def wait_gather(slot):
        pltpu.make_async_copy(g4_hbm.at[src_v.at[pl.ds(0, K)]],
                              gbufs[slot], gsems[slot]).wait()

    def issue_scatter(j, slot):
        pltpu.async_copy(sbufs[slot], acc_sh.at[dst_v.at[j]], ssems[slot],
                         add=True)

    def wait_scatter(slot):
        pltpu.make_async_copy(sbufs[slot], acc_sh.at[dst_v.at[0]],
                              ssems[slot]).wait()

    for p in range(2):
        @pl.loop(0, K)
        def _(r):
            for k in range(4):
                sb0[r, pl.ds(k * LN, LN)] = jnp.zeros((LN,), _f32)

        for qq in range(RPT // K):
            pltpu.sync_copy(sb0, acc_sh.at[pl.ds(s * RPT + qq * K, K)])

        pltpu.sync_copy(src_hbm.at[s], src_v)
        offv = jnp.full((LN,), c * (2 * NP) + p, _i32)

        @pl.loop(0, (CH1 * K) // LN)
        def _(i):
            v = src_v[pl.ds(i * LN, LN)]
            src_v[pl.ds(i * LN, LN)] = v + v + offv

        plsc.subcore_barrier()

        for slot in range(_NS):
            issue_gather(slot, slot)

        @pl.loop(0, _NR)
        def _(r):
            for slot in range(_NS):
                j = r * _NS + slot
                wait_gather(slot)

                @pl.when(r > 0)
                def _():
                    wait_scatter(slot)

                gb = gbufs[slot]
                sb = sbufs[slot]
                jb = j * K

                @plsc.parallel_loop(0, K, unroll=4)
                def _(e):
                    widx = jnp.full((LN,), jb + e, _i32)
                    wv = plsc.load_gather(w_v, [widx])
                    for k in range(4):
                        sb[e, pl.ds(k * LN, LN)] = gb[e, pl.ds(k * LN, LN)] * wv

                issue_scatter(j, slot)

                @pl.when(r < _NR - 1)
                def _():
                    issue_gather(j + _NS, slot)

        for slot in range(_NS):
            wait_scatter(slot)
        plsc.subcore_barrier()
        q = c * 2 + p
        pltpu.sync_copy(acc_sh.at[pl.ds(s * RPT, RPT)],
                        out_hbm.at[q].at[pl.ds(s * RPT, RPT)])


_agg1_call = pl.kernel(
    _agg1_body,
    out_type=jax.ShapeDtypeStruct((2 * NC, NP, 64), _f32),
    mesh=_mesh,
    scratch_types=[
        pltpu.VMEM((CH1 * K,), _i32),
        pltpu.VMEM((CH1, K), _i32),
        pltpu.VMEM((CH1 * K,), _f32),
        pltpu.VMEM((K, 64), _f32),
        pltpu.VMEM((K, 64), _f32),
        pltpu.VMEM((K, 64), _f32),
        pltpu.VMEM((K, 64), _f32),
        pltpu.VMEM_SHARED((NP, 64), _f32),
        pltpu.SemaphoreType.DMA,
        pltpu.SemaphoreType.DMA,
        pltpu.SemaphoreType.DMA,
        pltpu.SemaphoreType.DMA,
    ],
    compiler_params=_sc_params,
)


def _agg2_body(zg_hbm, src_hbm, dst_hbm, w_hbm, out_hbm,
               zg_v, src_v, dst_v, w_v, vals_v, zb_v, acc_sh, sem):
    c = lax.axis_index("c")
    s = lax.axis_index("s")
    wid = c * NT + s

    @pl.loop(0, RPT // LN)
    def _(i):
        zb_v[pl.ds(i * LN, LN)] = jnp.zeros((LN,), _f32)

    pltpu.sync_copy(zb_v, acc_sh.at[pl.ds(s * RPT, RPT)])

    pltpu.sync_copy(zg_hbm, zg_v)
    pltpu.sync_copy(src_hbm.at[wid], src_v)
    pltpu.sync_copy(dst_hbm.at[wid], dst_v)
    pltpu.sync_copy(w_hbm.at[wid], w_v)
    plsc.subcore_barrier()

    @pl.loop(0, CH2 * K // LN)
    def _(i):
        idx = src_v[pl.ds(i * LN, LN)]
        vals = plsc.load_gather(zg_v, [idx])
        vals_v[pl.ds(i * LN, LN)] = vals * w_v[pl.ds(i * LN, LN)]

    descs = []
    for j in range(CH2):
        descs.append(
            pltpu.async_copy(vals_v.at[pl.ds(j * K, K)],
                             acc_sh.at[dst_v.at[j]], sem, add=True))
    for d in descs:
        d.wait()
    plsc.subcore_barrier()
    pltpu.sync_copy(acc_sh.at[pl.ds(s * RPT, RPT)],
                    out_hbm.at[c].at[pl.ds(s * RPT, RPT)])


_agg2_call = pl.kernel(
    _agg2_body,
    out_type=jax.ShapeDtypeStruct((NC, NP), _f32),
    mesh=_mesh,
    scratch_types=[
        pltpu.VMEM((NP,), _f32),
        pltpu.VMEM((CH2 * K,), _i32),
        pltpu.VMEM((CH2, K), _i32),
        pltpu.VMEM((CH2 * K,), _f32),
        pltpu.VMEM((CH2 * K,), _f32),
        pltpu.VMEM((RPT,), _f32),
        pltpu.VMEM_SHARED((NP,), _f32),
        pltpu.SemaphoreType.DMA,
    ],
    compiler_params=_sc_params,
)


def _mm_body(x_ref, w_ref, o_ref):
    o_ref[...] = jnp.dot(x_ref[...], w_ref[...],
                         preferred_element_type=_f32)


def _b_body(degp_ref, h_ref, g2_ref, dis_ref):
    dp = degp_ref[...]
    dis = lax.rsqrt(dp[:, 0:1] + dp[:, 1:2] + 1.0)
    g2_ref[...] = h_ref[...] * dis
    dis_ref[...] = dis


def _c_body(a0_ref, a1_ref, a2_ref, a3_ref, glo_ref, ghi_ref, dis_ref,
            w2_ref, b1_ref, zg_ref):
    dis = dis_ref[...]
    alo = jnp.concatenate([a0_ref[0], a1_ref[0]], axis=1)
    ahi = jnp.concatenate([a2_ref[0], a3_ref[0]], axis=1)
    rlo = jnp.maximum(dis * (alo + glo_ref[...]) + b1_ref[:, 0:128],
                      0.0)
    rhi = jnp.maximum(dis * (ahi + ghi_ref[...]) + b1_ref[:, 128:256],
                      0.0)
    z = (jnp.dot(rlo, w2_ref[0:128, :], preferred_element_type=_f32)
         + jnp.dot(rhi, w2_ref[128:256, :], preferred_element_type=_f32))
    zg_ref[...] = dis * z


def _d_body(a2_ref, zg_ref, dis_ref, b2_ref, o_ref):
    a2 = a2_ref[0] + a2_ref[1]
    o_ref[...] = dis_ref[...] * (a2 + zg_ref[...]) + b2_ref[0, 0]


_BS = 1024
_GR = NP // _BS


def kernel(x, edge_index, edge_weight, W1, b1, W2, b2):
    src = edge_index[0]
    dst = edge_index[1]
    pad = EP - E
    src_p = jnp.concatenate([src, jnp.zeros((pad,), _i32)])
    dst_p = jnp.concatenate([dst, jnp.zeros((pad,), _i32)])
    w_p = jnp.concatenate([edge_weight, jnp.zeros((pad,), _f32)])
    x_p = jnp.pad(x, ((0, NP - N), (0, 0)))

    src_a = src_p.reshape(NT, CH1 * K)
    dst_a = dst_p.reshape(NT, CH1, K)
    w_a = w_p.reshape(NT, CH1 * K)
    src_d = src_p.reshape(NC * NT, CH2 * K)
    dst_d = dst_p.reshape(NC * NT, CH2, K)
    w_d = w_p.reshape(NC * NT, CH2, K)
    w_d_flat = w_p.reshape(NC * NT, CH2 * K)

    deg_p = _deg_call(dst_d, w_d)

    h = pl.pallas_call(
        _mm_body,
        grid=(_GR,),
        in_specs=[pl.BlockSpec((_BS, D), lambda i: (i, 0)),
                  pl.BlockSpec((D, D), lambda i: (0, 0))],
        out_specs=pl.BlockSpec((_BS, D), lambda i: (i, 0)),
        out_shape=jax.ShapeDtypeStruct((NP, D), _f32),
    )(x_p, W1)

    degp_t = deg_p.T
    g2, dis = pl.pallas_call(
        _b_body,
        grid=(_GR, 2),
        in_specs=[pl.BlockSpec((_BS, NC), lambda i, c: (i, 0)),
                  pl.BlockSpec((_BS, 128), lambda i, c: (i, c))],
        out_specs=[pl.BlockSpec((_BS, 128), lambda i, c: (c * _GR + i, 0)),
                   pl.BlockSpec((_BS, 1), lambda i, c: (i, 0))],
        out_shape=[jax.ShapeDtypeStruct((NC * NP, 128), _f32),
                   jax.ShapeDtypeStruct((NP, 1), _f32)],
    )(degp_t, h)

    acc = _agg1_call(g2.reshape(4 * NP, 64), src_a, dst_a, w_a)

    zg = pl.pallas_call(
        _c_body,
        grid=(_GR,),
        in_specs=[pl.BlockSpec((1, _BS, 64), lambda i: (0, i, 0)),
                  pl.BlockSpec((1, _BS, 64), lambda i: (1, i, 0)),
                  pl.BlockSpec((1, _BS, 64), lambda i: (2, i, 0)),
                  pl.BlockSpec((1, _BS, 64), lambda i: (3, i, 0)),
                  pl.BlockSpec((_BS, 128), lambda i: (i, 0)),
                  pl.BlockSpec((_BS, 128), lambda i: (_GR + i, 0)),
                  pl.BlockSpec((_BS, 1), lambda i: (i, 0)),
                  pl.BlockSpec((D, 1), lambda i: (0, 0)),
                  pl.BlockSpec((1, D), lambda i: (0, 0))],
        out_specs=pl.BlockSpec((_BS, 1), lambda i: (i, 0)),
        out_shape=jax.ShapeDtypeStruct((NP, 1), _f32),
    )(acc, acc, acc, acc, g2, g2, dis, W2, b1.reshape(1, D))

    acc2 = _agg2_call(zg.reshape(NP), src_d, dst_d, w_d_flat)

    out = pl.pallas_call(
        _d_body,
        out_shape=jax.ShapeDtypeStruct((NP // 128, 128), _f32),
    )(acc2.reshape(NC, NP // 128, 128), zg.reshape(NP // 128, 128),
      dis.reshape(NP // 128, 128), b2.reshape(1, 1))

    return out.reshape(NP)[:N]


# speedup vs baseline: 13.9356x; 1.1304x over previous
import dataclasses
import functools

import jax
import jax.numpy as jnp
from jax import lax
from jax.experimental import pallas as pl
from jax.experimental.pallas import tpu as pltpu
from jax.experimental.pallas import tpu_sc as plsc

N = 10000
D = 256
E = 160000

NP = 10240
EP = 163840
NC = 2
NT = 16
LN = 16
K = 128
CH1 = EP // NT // K
CH2 = EP // (NC * NT) // K
RPT = NP // NT

_mesh = plsc.VectorSubcoreMesh(core_axis_name="c", subcore_axis_name="s")

_sc_params = pltpu.CompilerParams(needs_layout_passes=False,
                                  use_tc_tiling_on_sc=False)

_f32 = jnp.float32
_i32 = jnp.int32


def _deg_body(dst_hbm, w_hbm, out_hbm, dst_v, w_v, zb_v, acc_sh, sem):
    c = lax.axis_index("c")
    s = lax.axis_index("s")
    wid = c * NT + s

    @pl.loop(0, RPT // LN)
    def _(i):
        zb_v[pl.ds(i * LN, LN)] = jnp.zeros((LN,), _f32)

    pltpu.sync_copy(zb_v, acc_sh.at[pl.ds(s * RPT, RPT)])
    plsc.subcore_barrier()

    pltpu.sync_copy(dst_hbm.at[wid], dst_v)
    pltpu.sync_copy(w_hbm.at[wid], w_v)

    descs = []
    for j in range(CH2):
        descs.append(
            pltpu.async_copy(w_v.at[j], acc_sh.at[dst_v.at[j]], sem, add=True))
    for d in descs:
        d.wait()
    plsc.subcore_barrier()
    pltpu.sync_copy(acc_sh.at[pl.ds(s * RPT, RPT)],
                    out_hbm.at[c].at[pl.ds(s * RPT, RPT)])


_deg_call = pl.kernel(
    _deg_body,
    out_type=jax.ShapeDtypeStruct((NC, NP), _f32),
    mesh=_mesh,
    scratch_types=[
        pltpu.VMEM((CH2, K), _i32),
        pltpu.VMEM((CH2, K), _f32),
        pltpu.VMEM((RPT,), _f32),
        pltpu.VMEM_SHARED((NP,), _f32),
        pltpu.SemaphoreType.DMA,
    ],
    compiler_params=_sc_params,
)


_B = 20
_NB = CH1 // _B


def _agg1_body(g2_hbm, src_hbm, dst_hbm, w_hbm, out_hbm,
               src_v, dst_v, w_v, gb0, gb1, acc_sh,
               gm0, gm1, sm0, sm1):
    c = lax.axis_index("c")
    s = lax.axis_index("s")
    gbufs = (gb0, gb1)
    gsems = (gm0, gm1)
    ssems = (sm0, sm1)

    @pl.loop(0, K)
    def _(r):
        for k in range(8):
            gb0[r, pl.ds(k * LN, LN)] = jnp.zeros((LN,), _f32)

    for qq in range(RPT // K):
        pltpu.sync_copy(gb0, acc_sh.at[pl.ds(s * RPT + qq * K, K)])
    plsc.subcore_barrier()

    offv = jnp.full((LN,), c * NP, _i32)

    def issue_gather(j, slot):
        pltpu.async_copy(g2_hbm.at[src_v.at[pl.ds(j * K, K)]],
                         gbufs[slot], gsems[slot])

    def wait_gather(slot):
        pltpu.make_async_copy(g2_hbm.at[src_v.at[pl.ds(0, K)]],
                              gbufs[slot], gsems[slot]).wait()

    def issue_scatter(j, slot):
        pltpu.async_copy(gbufs[slot], acc_sh.at[dst_v.at[j]], ssems[slot],
                         add=True)

    def wait_scatter(slot):
        pltpu.make_async_copy(gbufs[slot], acc_sh.at[dst_v.at[0]],
                              ssems[slot]).wait()

    for b in range(_NB):
        pltpu.sync_copy(src_hbm.at[s].at[b], src_v)
        pltpu.sync_copy(dst_hbm.at[s].at[b], dst_v)
        pltpu.sync_copy(w_hbm.at[s].at[b], w_v)

        @pl.loop(0, (_B * K) // LN)
        def _(i):
            src_v[pl.ds(i * LN, LN)] = src_v[pl.ds(i * LN, LN)] + offv

        issue_gather(0, 0)
        issue_gather(1, 1)

        @pl.loop(0, _B // 2)
        def _(r):
            for slot in range(2):
                j = r * 2 + slot
                wait_gather(slot)
                gb = gbufs[slot]
                jb = j * K

                @plsc.parallel_loop(0, K, unroll=4)
                def _(e):
                    widx = jnp.full((LN,), jb + e, _i32)
                    wv = plsc.load_gather(w_v, [widx])
                    for k in range(8):
                        gb[e, pl.ds(k * LN, LN)] = gb[e, pl.ds(k * LN, LN)] * wv

                issue_scatter(j, slot)

                @pl.when(r < _B // 2 - 1)
                def _():
                    wait_scatter(slot)
                    issue_gather(j + 2, slot)

        wait_scatter(0)
        wait_scatter(1)

    plsc.subcore_barrier()
    pltpu.sync_copy(acc_sh.at[pl.ds(s * RPT, RPT)],
                    out_hbm.at[c].at[pl.ds(s * RPT, RPT)])


_agg1_call = pl.kernel(
    _agg1_body,
    out_type=jax.ShapeDtypeStruct((NC, NP, 128), _f32),
    mesh=_mesh,
    scratch_types=[
        pltpu.VMEM((_B * K,), _i32),
        pltpu.VMEM((_B, K), _i32),
        pltpu.VMEM((_B * K,), _f32),
        pltpu.VMEM((K, 128), _f32),
        pltpu.VMEM((K, 128), _f32),
        pltpu.VMEM_SHARED((NP, 128), _f32),
        pltpu.SemaphoreType.DMA,
        pltpu.SemaphoreType.DMA,
        pltpu.SemaphoreType.DMA,
        pltpu.SemaphoreType.DMA,
    ],
    compiler_params=_sc_params,
)


def _agg2_body(zg_hbm, src_hbm, dst_hbm, w_hbm, out_hbm,
               zg_v, src_v, dst_v, w_v, vals_v, zb_v, acc_sh, sem):
    c = lax.axis_index("c")
    s = lax.axis_index("s")
    wid = c * NT + s

    @pl.loop(0, RPT // LN)
    def _(i):
        zb_v[pl.ds(i * LN, LN)] = jnp.zeros((LN,), _f32)

    pltpu.sync_copy(zb_v, acc_sh.at[pl.ds(s * RPT, RPT)])

    pltpu.sync_copy(zg_hbm, zg_v)
    pltpu.sync_copy(src_hbm.at[wid], src_v)
    pltpu.sync_copy(dst_hbm.at[wid], dst_v)
    pltpu.sync_copy(w_hbm.at[wid], w_v)
    plsc.subcore_barrier()

    @pl.loop(0, CH2 * K // LN)
    def _(i):
        idx = src_v[pl.ds(i * LN, LN)]
        vals = plsc.load_gather(zg_v, [idx])
        vals_v[pl.ds(i * LN, LN)] = vals * w_v[pl.ds(i * LN, LN)]

    descs = []
    for j in range(CH2):
        descs.append(
            pltpu.async_copy(vals_v.at[pl.ds(j * K, K)],
                             acc_sh.at[dst_v.at[j]], sem, add=True))
    for d in descs:
        d.wait()
    plsc.subcore_barrier()
    pltpu.sync_copy(acc_sh.at[pl.ds(s * RPT, RPT)],
                    out_hbm.at[c].at[pl.ds(s * RPT, RPT)])


_agg2_call = pl.kernel(
    _agg2_body,
    out_type=jax.ShapeDtypeStruct((NC, NP), _f32),
    mesh=_mesh,
    scratch_types=[
        pltpu.VMEM((NP,), _f32),
        pltpu.VMEM((CH2 * K,), _i32),
        pltpu.VMEM((CH2, K), _i32),
        pltpu.VMEM((CH2 * K,), _f32),
        pltpu.VMEM((CH2 * K,), _f32),
        pltpu.VMEM((RPT,), _f32),
        pltpu.VMEM_SHARED((NP,), _f32),
        pltpu.SemaphoreType.DMA,
    ],
    compiler_params=_sc_params,
)


def _mm_body(x_ref, w_ref, o_ref):
    o_ref[...] = jnp.dot(x_ref[...], w_ref[...],
                         preferred_element_type=_f32)


def _b_body(degp_ref, h_ref, g2_ref, dis_ref):
    dp = degp_ref[...]
    dis = lax.rsqrt(dp[:, 0:1] + dp[:, 1:2] + 1.0)
    g2_ref[...] = h_ref[...] * dis
    dis_ref[...] = dis


def _c_body(alo_ref, ahi_ref, glo_ref, ghi_ref, dis_ref,
            w2_ref, b1_ref, zg_ref):
    dis = dis_ref[...]
    rlo = jnp.maximum(dis * (alo_ref[0] + glo_ref[...]) + b1_ref[:, 0:128],
                      0.0)
    rhi = jnp.maximum(dis * (ahi_ref[0] + ghi_ref[...]) + b1_ref[:, 128:256],
                      0.0)
    z = (jnp.dot(rlo, w2_ref[0:128, :], preferred_element_type=_f32)
         + jnp.dot(rhi, w2_ref[128:256, :], preferred_element_type=_f32))
    zg_ref[...] = dis * z


def _d_body(a2_ref, zg_ref, dis_ref, b2_ref, o_ref):
    a2 = a2_ref[0] + a2_ref[1]
    o_ref[...] = dis_ref[...] * (a2 + zg_ref[...]) + b2_ref[0, 0]


_BS = 1024
_GR = NP // _BS


def kernel(x, edge_index, edge_weight, W1, b1, W2, b2):
    src = edge_index[0]
    dst = edge_index[1]
    pad = EP - E
    src_p = jnp.concatenate([src, jnp.zeros((pad,), _i32)])
    dst_p = jnp.concatenate([dst, jnp.zeros((pad,), _i32)])
    w_p = jnp.concatenate([edge_weight, jnp.zeros((pad,), _f32)])
    x_p = jnp.pad(x, ((0, NP - N), (0, 0)))

    src_a = src_p.reshape(NT, _NB, _B * K)
    dst_a = dst_p.reshape(NT, _NB, _B, K)
    w_a = w_p.reshape(NT, _NB, _B * K)
    src_d = src_p.reshape(NC * NT, CH2 * K)
    dst_d = dst_p.reshape(NC * NT, CH2, K)
    w_d = w_p.reshape(NC * NT, CH2, K)
    w_d_flat = w_p.reshape(NC * NT, CH2 * K)

    deg_p = _deg_call(dst_d, w_d)

    h = pl.pallas_call(
        _mm_body,
        grid=(_GR,),
        in_specs=[pl.BlockSpec((_BS, D), lambda i: (i, 0)),
                  pl.BlockSpec((D, D), lambda i: (0, 0))],
        out_specs=pl.BlockSpec((_BS, D), lambda i: (i, 0)),
        out_shape=jax.ShapeDtypeStruct((NP, D), _f32),
    )(x_p, W1)

    degp_t = deg_p.T
    g2, dis = pl.pallas_call(
        _b_body,
        grid=(_GR, 2),
        in_specs=[pl.BlockSpec((_BS, NC), lambda i, c: (i, 0)),
                  pl.BlockSpec((_BS, 128), lambda i, c: (i, c))],
        out_specs=[pl.BlockSpec((_BS, 128), lambda i, c: (c * _GR + i, 0)),
                   pl.BlockSpec((_BS, 1), lambda i, c: (i, 0))],
        out_shape=[jax.ShapeDtypeStruct((NC * NP, 128), _f32),
                   jax.ShapeDtypeStruct((NP, 1), _f32)],
    )(degp_t, h)

    acc = _agg1_call(g2, src_a, dst_a, w_a)

    zg = pl.pallas_call(
        _c_body,
        grid=(_GR,),
        in_specs=[pl.BlockSpec((1, _BS, 128), lambda i: (0, i, 0)),
                  pl.BlockSpec((1, _BS, 128), lambda i: (1, i, 0)),
                  pl.BlockSpec((_BS, 128), lambda i: (i, 0)),
                  pl.BlockSpec((_BS, 128), lambda i: (_GR + i, 0)),
                  pl.BlockSpec((_BS, 1), lambda i: (i, 0)),
                  pl.BlockSpec((D, 1), lambda i: (0, 0)),
                  pl.BlockSpec((1, D), lambda i: (0, 0))],
        out_specs=pl.BlockSpec((_BS, 1), lambda i: (i, 0)),
        out_shape=jax.ShapeDtypeStruct((NP, 1), _f32),
    )(acc, acc, g2, g2, dis, W2, b1.reshape(1, D))

    acc2 = _agg2_call(zg.reshape(NP), src_d, dst_d, w_d_flat)

    out = pl.pallas_call(
        _d_body,
        out_shape=jax.ShapeDtypeStruct((NP // 128, 128), _f32),
    )(acc2.reshape(NC, NP // 128, 128), zg.reshape(NP // 128, 128),
      dis.reshape(NP // 128, 128), b2.reshape(1, 1))

    return out.reshape(NP)[:N]


# speedup vs baseline: 13.9803x; 1.0032x over previous
import dataclasses
import functools

import jax
import jax.numpy as jnp
from jax import lax
from jax.experimental import pallas as pl
from jax.experimental.pallas import tpu as pltpu
from jax.experimental.pallas import tpu_sc as plsc

N = 10000
D = 256
E = 160000

NP = 10240
EP = 163840
NC = 2
NT = 16
LN = 16
K = 128
CH1 = EP // NT // K
CH2 = EP // (NC * NT) // K
RPT = NP // NT

_mesh = plsc.VectorSubcoreMesh(core_axis_name="c", subcore_axis_name="s")

_sc_params = pltpu.CompilerParams(needs_layout_passes=False,
                                  use_tc_tiling_on_sc=False)

_f32 = jnp.float32
_i32 = jnp.int32


def _deg_body(dst_hbm, w_hbm, out_hbm, dst_v, w_v, zb_v, acc_sh, sem):
    c = lax.axis_index("c")
    s = lax.axis_index("s")
    wid = c * NT + s

    @pl.loop(0, RPT // LN)
    def _(i):
        zb_v[pl.ds(i * LN, LN)] = jnp.zeros((LN,), _f32)

    pltpu.sync_copy(zb_v, acc_sh.at[pl.ds(s * RPT, RPT)])
    plsc.subcore_barrier()

    pltpu.sync_copy(dst_hbm.at[wid], dst_v)
    pltpu.sync_copy(w_hbm.at[wid], w_v)

    descs = []
    for j in range(CH2):
        descs.append(
            pltpu.async_copy(w_v.at[j], acc_sh.at[dst_v.at[j]], sem, add=True))
    for d in descs:
        d.wait()
    plsc.subcore_barrier()
    pltpu.sync_copy(acc_sh.at[pl.ds(s * RPT, RPT)],
                    out_hbm.at[c].at[pl.ds(s * RPT, RPT)])


_deg_call = pl.kernel(
    _deg_body,
    out_type=jax.ShapeDtypeStruct((NC, NP), _f32),
    mesh=_mesh,
    scratch_types=[
        pltpu.VMEM((CH2, K), _i32),
        pltpu.VMEM((CH2, K), _f32),
        pltpu.VMEM((RPT,), _f32),
        pltpu.VMEM_SHARED((NP,), _f32),
        pltpu.SemaphoreType.DMA,
    ],
    compiler_params=_sc_params,
)


K1 = 64
CHA = EP // NT // K1
_B = 40
_NB = CHA // _B


def _agg1_body(g2_hbm, src_hbm, dst_hbm, w_hbm, out_hbm,
               src_v, dst_v, w_v, gb0, gb1, sb0, sb1, acc_sh,
               gm0, gm1, sm0, sm1):
    c = lax.axis_index("c")
    s = lax.axis_index("s")
    gbufs = (gb0, gb1)
    sbufs = (sb0, sb1)
    gsems = (gm0, gm1)
    ssems = (sm0, sm1)

    @pl.loop(0, K1)
    def _(r):
        for k in range(8):
            sb0[r, pl.ds(k * LN, LN)] = jnp.zeros((LN,), _f32)

    for qq in range(RPT // K1):
        pltpu.sync_copy(sb0, acc_sh.at[pl.ds(s * RPT + qq * K1, K1)])
    plsc.subcore_barrier()

    offv = jnp.full((LN,), c * NP, _i32)

    def issue_gather(j, slot):
        pltpu.async_copy(g2_hbm.at[src_v.at[pl.ds(j * K1, K1)]],
                         gbufs[slot], gsems[slot])

    def wait_gather(slot):
        pltpu.make_async_copy(g2_hbm.at[src_v.at[pl.ds(0, K1)]],
                              gbufs[slot], gsems[slot]).wait()

    def issue_scatter(j, slot):
        pltpu.async_copy(sbufs[slot], acc_sh.at[dst_v.at[j]], ssems[slot],
                         add=True)

    def wait_scatter(slot):
        pltpu.make_async_copy(sbufs[slot], acc_sh.at[dst_v.at[0]],
                              ssems[slot]).wait()

    for b in range(_NB):
        pltpu.sync_copy(src_hbm.at[s].at[b], src_v)
        pltpu.sync_copy(dst_hbm.at[s].at[b], dst_v)
        pltpu.sync_copy(w_hbm.at[s].at[b], w_v)

        @pl.loop(0, (_B * K1) // LN)
        def _(i):
            src_v[pl.ds(i * LN, LN)] = src_v[pl.ds(i * LN, LN)] + offv

        issue_gather(0, 0)
        issue_gather(1, 1)

        @pl.loop(0, _B // 2)
        def _(r):
            for slot in range(2):
                j = r * 2 + slot
                wait_gather(slot)

                @pl.when(r > 0)
                def _():
                    wait_scatter(slot)

                gb = gbufs[slot]
                sb = sbufs[slot]
                jb = j * K1

                @plsc.parallel_loop(0, K1, unroll=4)
                def _(e):
                    widx = jnp.full((LN,), jb + e, _i32)
                    wv = plsc.load_gather(w_v, [widx])
                    for k in range(8):
                        sb[e, pl.ds(k * LN, LN)] = gb[e, pl.ds(k * LN, LN)] * wv

                issue_scatter(j, slot)

                @pl.when(r < _B // 2 - 1)
                def _():
                    issue_gather(j + 2, slot)

        wait_scatter(0)
        wait_scatter(1)

    plsc.subcore_barrier()
    pltpu.sync_copy(acc_sh.at[pl.ds(s * RPT, RPT)],
                    out_hbm.at[c].at[pl.ds(s * RPT, RPT)])


_agg1_call = pl.kernel(
    _agg1_body,
    out_type=jax.ShapeDtypeStruct((NC, NP, 128), _f32),
    mesh=_mesh,
    scratch_types=[
        pltpu.VMEM((_B * K1,), _i32),
        pltpu.VMEM((_B, K1), _i32),
        pltpu.VMEM((_B * K1,), _f32),
        pltpu.VMEM((K1, 128), _f32),
        pltpu.VMEM((K1, 128), _f32),
        pltpu.VMEM((K1, 128), _f32),
        pltpu.VMEM((K1, 128), _f32),
        pltpu.VMEM_SHARED((NP, 128), _f32),
        pltpu.SemaphoreType.DMA,
        pltpu.SemaphoreType.DMA,
        pltpu.SemaphoreType.DMA,
        pltpu.SemaphoreType.DMA,
    ],
    compiler_params=_sc_params,
)


def _agg2_body(zg_hbm, src_hbm, dst_hbm, w_hbm, out_hbm,
               zg_v, src_v, dst_v, w_v, vals_v, zb_v, acc_sh, sem):
    c = lax.axis_index("c")
    s = lax.axis_index("s")
    wid = c * NT + s

    @pl.loop(0, RPT // LN)
    def _(i):
        zb_v[pl.ds(i * LN, LN)] = jnp.zeros((LN,), _f32)

    pltpu.sync_copy(zb_v, acc_sh.at[pl.ds(s * RPT, RPT)])

    pltpu.sync_copy(zg_hbm, zg_v)
    pltpu.sync_copy(src_hbm.at[wid], src_v)
    pltpu.sync_copy(dst_hbm.at[wid], dst_v)
    pltpu.sync_copy(w_hbm.at[wid], w_v)
    plsc.subcore_barrier()

    @pl.loop(0, CH2 * K // LN)
    def _(i):
        idx = src_v[pl.ds(i * LN, LN)]
        vals = plsc.load_gather(zg_v, [idx])
        vals_v[pl.ds(i * LN, LN)] = vals * w_v[pl.ds(i * LN, LN)]

    descs = []
    for j in range(CH2):
        descs.append(
            pltpu.async_copy(vals_v.at[pl.ds(j * K, K)],
                             acc_sh.at[dst_v.at[j]], sem, add=True))
    for d in descs:
        d.wait()
    plsc.subcore_barrier()
    pltpu.sync_copy(acc_sh.at[pl.ds(s * RPT, RPT)],
                    out_hbm.at[c].at[pl.ds(s * RPT, RPT)])


_agg2_call = pl.kernel(
    _agg2_body,
    out_type=jax.ShapeDtypeStruct((NC, NP), _f32),
    mesh=_mesh,
    scratch_types=[
        pltpu.VMEM((NP,), _f32),
        pltpu.VMEM((CH2 * K,), _i32),
        pltpu.VMEM((CH2, K), _i32),
        pltpu.VMEM((CH2 * K,), _f32),
        pltpu.VMEM((CH2 * K,), _f32),
        pltpu.VMEM((RPT,), _f32),
        pltpu.VMEM_SHARED((NP,), _f32),
        pltpu.SemaphoreType.DMA,
    ],
    compiler_params=_sc_params,
)


def _mm_body(x_ref, w_ref, o_ref):
    o_ref[...] = jnp.dot(x_ref[...], w_ref[...],
                         preferred_element_type=_f32)


def _b_body(degp_ref, h_ref, g2_ref, dis_ref):
    dp = degp_ref[...]
    dis = lax.rsqrt(dp[:, 0:1] + dp[:, 1:2] + 1.0)
    g2_ref[...] = h_ref[...] * dis
    dis_ref[...] = dis


def _c_body(alo_ref, ahi_ref, glo_ref, ghi_ref, dis_ref,
            w2_ref, b1_ref, zg_ref):
    dis = dis_ref[...]
    rlo = jnp.maximum(dis * (alo_ref[0] + glo_ref[...]) + b1_ref[:, 0:128],
                      0.0)
    rhi = jnp.maximum(dis * (ahi_ref[0] + ghi_ref[...]) + b1_ref[:, 128:256],
                      0.0)
    z = (jnp.dot(rlo, w2_ref[0:128, :], preferred_element_type=_f32)
         + jnp.dot(rhi, w2_ref[128:256, :], preferred_element_type=_f32))
    zg_ref[...] = dis * z


def _d_body(a2_ref, zg_ref, dis_ref, b2_ref, o_ref):
    a2 = a2_ref[0] + a2_ref[1]
    o_ref[...] = dis_ref[...] * (a2 + zg_ref[...]) + b2_ref[0, 0]


_BS = 1024
_GR = NP // _BS


def kernel(x, edge_index, edge_weight, W1, b1, W2, b2):
    src = edge_index[0]
    dst = edge_index[1]
    pad = EP - E
    src_p = jnp.concatenate([src, jnp.zeros((pad,), _i32)])
    dst_p = jnp.concatenate([dst, jnp.zeros((pad,), _i32)])
    w_p = jnp.concatenate([edge_weight, jnp.zeros((pad,), _f32)])
    x_p = jnp.pad(x, ((0, NP - N), (0, 0)))

    src_a = src_p.reshape(NT, _NB, _B * K1)
    dst_a = dst_p.reshape(NT, _NB, _B, K1)
    w_a = w_p.reshape(NT, _NB, _B * K1)
    src_d = src_p.reshape(NC * NT, CH2 * K)
    dst_d = dst_p.reshape(NC * NT, CH2, K)
    w_d = w_p.reshape(NC * NT, CH2, K)
    w_d_flat = w_p.reshape(NC * NT, CH2 * K)

    deg_p = _deg_call(dst_d, w_d)

    h = pl.pallas_call(
        _mm_body,
        grid=(_GR,),
        in_specs=[pl.BlockSpec((_BS, D), lambda i: (i, 0)),
                  pl.BlockSpec((D, D), lambda i: (0, 0))],
        out_specs=pl.BlockSpec((_BS, D), lambda i: (i, 0)),
        out_shape=jax.ShapeDtypeStruct((NP, D), _f32),
    )(x_p, W1)

    degp_t = deg_p.T
    g2, dis = pl.pallas_call(
        _b_body,
        grid=(_GR, 2),
        in_specs=[pl.BlockSpec((_BS, NC), lambda i, c: (i, 0)),
                  pl.BlockSpec((_BS, 128), lambda i, c: (i, c))],
        out_specs=[pl.BlockSpec((_BS, 128), lambda i, c: (c * _GR + i, 0)),
                   pl.BlockSpec((_BS, 1), lambda i, c: (i, 0))],
        out_shape=[jax.ShapeDtypeStruct((NC * NP, 128), _f32),
                   jax.ShapeDtypeStruct((NP, 1), _f32)],
    )(degp_t, h)

    acc = _agg1_call(g2, src_a, dst_a, w_a)

    zg = pl.pallas_call(
        _c_body,
        grid=(_GR,),
        in_specs=[pl.BlockSpec((1, _BS, 128), lambda i: (0, i, 0)),
                  pl.BlockSpec((1, _BS, 128), lambda i: (1, i, 0)),
                  pl.BlockSpec((_BS, 128), lambda i: (i, 0)),
                  pl.BlockSpec((_BS, 128), lambda i: (_GR + i, 0)),
                  pl.BlockSpec((_BS, 1), lambda i: (i, 0)),
                  pl.BlockSpec((D, 1), lambda i: (0, 0)),
                  pl.BlockSpec((1, D), lambda i: (0, 0))],
        out_specs=pl.BlockSpec((_BS, 1), lambda i: (i, 0)),
        out_shape=jax.ShapeDtypeStruct((NP, 1), _f32),
    )(acc, acc, g2, g2, dis, W2, b1.reshape(1, D))

    acc2 = _agg2_call(zg.reshape(NP), src_d, dst_d, w_d_flat)

    out = pl.pallas_call(
        _d_body,
        out_shape=jax.ShapeDtypeStruct((NP // 128, 128), _f32),
    )(acc2.reshape(NC, NP // 128, 128), zg.reshape(NP // 128, 128),
      dis.reshape(NP // 128, 128), b2.reshape(1, 1))

    return out.reshape(NP)[:N]


# speedup vs baseline: 16.8248x; 1.2035x over previous
import dataclasses
import functools

import jax
import jax.numpy as jnp
from jax import lax
from jax.experimental import pallas as pl
from jax.experimental.pallas import tpu as pltpu
from jax.experimental.pallas import tpu_sc as plsc

N = 10000
D = 256
E = 160000

NP = 10240
EP = 163840
NC = 2
NT = 16
LN = 16
K = 128
CH1 = EP // NT // K
CH2 = EP // (NC * NT) // K
RPT = NP // NT

_mesh = plsc.VectorSubcoreMesh(core_axis_name="c", subcore_axis_name="s")

_sc_params = pltpu.CompilerParams(needs_layout_passes=False,
                                  use_tc_tiling_on_sc=False)

_f32 = jnp.float32
_i32 = jnp.int32


def _deg_body(dst_hbm, w_hbm, out_hbm, dst_v, w_v, zb_v, acc_sh, sem):
    c = lax.axis_index("c")
    s = lax.axis_index("s")
    wid = c * NT + s

    @pl.loop(0, RPT // LN)
    def _(i):
        zb_v[pl.ds(i * LN, LN)] = jnp.zeros((LN,), _f32)

    pltpu.sync_copy(zb_v, acc_sh.at[pl.ds(s * RPT, RPT)])
    plsc.subcore_barrier()

    pltpu.sync_copy(dst_hbm.at[wid], dst_v)
    pltpu.sync_copy(w_hbm.at[wid], w_v)

    descs = []
    for j in range(CH2):
        descs.append(
            pltpu.async_copy(w_v.at[j], acc_sh.at[dst_v.at[j]], sem, add=True))
    for d in descs:
        d.wait()
    plsc.subcore_barrier()
    pltpu.sync_copy(acc_sh.at[pl.ds(s * RPT, RPT)],
                    out_hbm.at[c].at[pl.ds(s * RPT, RPT)])


_deg_call = pl.kernel(
    _deg_body,
    out_type=jax.ShapeDtypeStruct((NC, NP), _f32),
    mesh=_mesh,
    scratch_types=[
        pltpu.VMEM((CH2, K), _i32),
        pltpu.VMEM((CH2, K), _f32),
        pltpu.VMEM((RPT,), _f32),
        pltpu.VMEM_SHARED((NP,), _f32),
        pltpu.SemaphoreType.DMA,
    ],
    compiler_params=_sc_params,
)


CHQ = EP // NT // K
_B = 20
_NB = CHQ // _B


def _agg1_body(g2_hbm, src_hbm, dst_hbm, w_hbm, out_hbm,
               src_v, dst_v, w_v, gb0, gb1, sb0, sb1, g_sh, acc_sh,
               gm0, gm1, sm0, sm1):
    c = lax.axis_index("c")
    s = lax.axis_index("s")
    gbufs = (gb0, gb1)
    sbufs = (sb0, sb1)
    gsems = (gm0, gm1)
    ssems = (sm0, sm1)

    def issue_gather(j, slot):
        pltpu.async_copy(g_sh.at[src_v.at[pl.ds(j * K, K)]],
                         gbufs[slot], gsems[slot])

    def wait_gather(slot):
        pltpu.make_async_copy(g_sh.at[src_v.at[pl.ds(0, K)]],
                              gbufs[slot], gsems[slot]).wait()

    def issue_scatter(j, slot):
        pltpu.async_copy(sbufs[slot], acc_sh.at[dst_v.at[j]], ssems[slot],
                         add=True)

    def wait_scatter(slot):
        pltpu.make_async_copy(sbufs[slot], acc_sh.at[dst_v.at[0]],
                              ssems[slot]).wait()

    for p in range(2):
        @pl.loop(0, K)
        def _(r):
            for k in range(4):
                sb0[r, pl.ds(k * LN, LN)] = jnp.zeros((LN,), _f32)

        pltpu.sync_copy(
            g2_hbm.at[pl.ds(c * NP + s * RPT, RPT), pl.ds(p * 64, 64)],
            g_sh.at[pl.ds(s * RPT, RPT)])
        for qq in range(RPT // K):
            pltpu.sync_copy(sb0, acc_sh.at[pl.ds(s * RPT + qq * K, K)])
        plsc.subcore_barrier()

        for b in range(_NB):
            pltpu.sync_copy(src_hbm.at[s].at[b], src_v)
            pltpu.sync_copy(dst_hbm.at[s].at[b], dst_v)
            pltpu.sync_copy(w_hbm.at[s].at[b], w_v)

            issue_gather(0, 0)
            issue_gather(1, 1)

            @pl.loop(0, _B // 2)
            def _(r):
                for slot in range(2):
                    j = r * 2 + slot
                    wait_gather(slot)

                    @pl.when(r > 0)
                    def _():
                        wait_scatter(slot)

                    gb = gbufs[slot]
                    sb = sbufs[slot]
                    jb = j * K

                    @plsc.parallel_loop(0, K, unroll=4)
                    def _(e):
                        widx = jnp.full((LN,), jb + e, _i32)
                        wv = plsc.load_gather(w_v, [widx])
                        for k in range(4):
                            sb[e, pl.ds(k * LN, LN)] = gb[e, pl.ds(k * LN, LN)] * wv

                    issue_scatter(j, slot)

                    @pl.when(r < _B // 2 - 1)
                    def _():
                        issue_gather(j + 2, slot)

            wait_scatter(0)
            wait_scatter(1)

        plsc.subcore_barrier()
        pltpu.sync_copy(acc_sh.at[pl.ds(s * RPT, RPT)],
                        out_hbm.at[c * 2 + p].at[pl.ds(s * RPT, RPT)])


_agg1_call = pl.kernel(
    _agg1_body,
    out_type=jax.ShapeDtypeStruct((2 * NC, NP, 64), _f32),
    mesh=_mesh,
    scratch_types=[
        pltpu.VMEM((_B * K,), _i32),
        pltpu.VMEM((_B, K), _i32),
        pltpu.VMEM((_B * K,), _f32),
        pltpu.VMEM((K, 64), _f32),
        pltpu.VMEM((K, 64), _f32),
        pltpu.VMEM((K, 64), _f32),
        pltpu.VMEM((K, 64), _f32),
        pltpu.VMEM_SHARED((NP, 64), _f32),
        pltpu.VMEM_SHARED((NP, 64), _f32),
        pltpu.SemaphoreType.DMA,
        pltpu.SemaphoreType.DMA,
        pltpu.SemaphoreType.DMA,
        pltpu.SemaphoreType.DMA,
    ],
    compiler_params=_sc_params,
)


def _agg2_body(zg_hbm, src_hbm, dst_hbm, w_hbm, out_hbm,
               zg_v, src_v, dst_v, w_v, vals_v, zb_v, acc_sh, sem):
    c = lax.axis_index("c")
    s = lax.axis_index("s")
    wid = c * NT + s

    @pl.loop(0, RPT // LN)
    def _(i):
        zb_v[pl.ds(i * LN, LN)] = jnp.zeros((LN,), _f32)

    pltpu.sync_copy(zb_v, acc_sh.at[pl.ds(s * RPT, RPT)])

    pltpu.sync_copy(zg_hbm, zg_v)
    pltpu.sync_copy(src_hbm.at[wid], src_v)
    pltpu.sync_copy(dst_hbm.at[wid], dst_v)
    pltpu.sync_copy(w_hbm.at[wid], w_v)
    plsc.subcore_barrier()

    @pl.loop(0, CH2 * K // LN)
    def _(i):
        idx = src_v[pl.ds(i * LN, LN)]
        vals = plsc.load_gather(zg_v, [idx])
        vals_v[pl.ds(i * LN, LN)] = vals * w_v[pl.ds(i * LN, LN)]

    descs = []
    for j in range(CH2):
        descs.append(
            pltpu.async_copy(vals_v.at[pl.ds(j * K, K)],
                             acc_sh.at[dst_v.at[j]], sem, add=True))
    for d in descs:
        d.wait()
    plsc.subcore_barrier()
    pltpu.sync_copy(acc_sh.at[pl.ds(s * RPT, RPT)],
                    out_hbm.at[c].at[pl.ds(s * RPT, RPT)])


_agg2_call = pl.kernel(
    _agg2_body,
    out_type=jax.ShapeDtypeStruct((NC, NP), _f32),
    mesh=_mesh,
    scratch_types=[
        pltpu.VMEM((NP,), _f32),
        pltpu.VMEM((CH2 * K,), _i32),
        pltpu.VMEM((CH2, K), _i32),
        pltpu.VMEM((CH2 * K,), _f32),
        pltpu.VMEM((CH2 * K,), _f32),
        pltpu.VMEM((RPT,), _f32),
        pltpu.VMEM_SHARED((NP,), _f32),
        pltpu.SemaphoreType.DMA,
    ],
    compiler_params=_sc_params,
)


def _mm_body(x_ref, w_ref, o_ref):
    o_ref[...] = jnp.dot(x_ref[...], w_ref[...],
                         preferred_element_type=_f32)


def _b_body(degp_ref, h_ref, g2_ref, dis_ref):
    dp = degp_ref[...]
    dis = lax.rsqrt(dp[:, 0:1] + dp[:, 1:2] + 1.0)
    g2_ref[...] = h_ref[...] * dis
    dis_ref[...] = dis


def _c_body(a0_ref, a1_ref, a2_ref, a3_ref, glo_ref, ghi_ref, dis_ref,
            w2_ref, b1_ref, zg_ref):
    dis = dis_ref[...]
    alo = jnp.concatenate([a0_ref[0], a1_ref[0]], axis=1)
    ahi = jnp.concatenate([a2_ref[0], a3_ref[0]], axis=1)
    rlo = jnp.maximum(dis * (alo + glo_ref[...]) + b1_ref[:, 0:128],
                      0.0)
    rhi = jnp.maximum(dis * (ahi + ghi_ref[...]) + b1_ref[:, 128:256],
                      0.0)
    z = (jnp.dot(rlo, w2_ref[0:128, :], preferred_element_type=_f32)
         + jnp.dot(rhi, w2_ref[128:256, :], preferred_element_type=_f32))
    zg_ref[...] = dis * z


def _d_body(a2_ref, zg_ref, dis_ref, b2_ref, o_ref):
    a2 = a2_ref[0] + a2_ref[1]
    o_ref[...] = dis_ref[...] * (a2 + zg_ref[...]) + b2_ref[0, 0]


_BS = 1024
_GR = NP // _BS


def kernel(x, edge_index, edge_weight, W1, b1, W2, b2):
    src = edge_index[0]
    dst = edge_index[1]
    pad = EP - E
    src_p = jnp.concatenate([src, jnp.zeros((pad,), _i32)])
    dst_p = jnp.concatenate([dst, jnp.zeros((pad,), _i32)])
    w_p = jnp.concatenate([edge_weight, jnp.zeros((pad,), _f32)])
    x_p = jnp.pad(x, ((0, NP - N), (0, 0)))

    src_a = src_p.reshape(NT, _NB, _B * K)
    dst_a = dst_p.reshape(NT, _NB, _B, K)
    w_a = w_p.reshape(NT, _NB, _B * K)
    src_d = src_p.reshape(NC * NT, CH2 * K)
    dst_d = dst_p.reshape(NC * NT, CH2, K)
    w_d = w_p.reshape(NC * NT, CH2, K)
    w_d_flat = w_p.reshape(NC * NT, CH2 * K)

    deg_p = _deg_call(dst_d, w_d)

    h = pl.pallas_call(
        _mm_body,
        grid=(_GR,),
        in_specs=[pl.BlockSpec((_BS, D), lambda i: (i, 0)),
                  pl.BlockSpec((D, D), lambda i: (0, 0))],
        out_specs=pl.BlockSpec((_BS, D), lambda i: (i, 0)),
        out_shape=jax.ShapeDtypeStruct((NP, D), _f32),
    )(x_p, W1)

    degp_t = deg_p.T
    g2, dis = pl.pallas_call(
        _b_body,
        grid=(_GR, 2),
        in_specs=[pl.BlockSpec((_BS, NC), lambda i, c: (i, 0)),
                  pl.BlockSpec((_BS, 128), lambda i, c: (i, c))],
        out_specs=[pl.BlockSpec((_BS, 128), lambda i, c: (c * _GR + i, 0)),
                   pl.BlockSpec((_BS, 1), lambda i, c: (i, 0))],
        out_shape=[jax.ShapeDtypeStruct((NC * NP, 128), _f32),
                   jax.ShapeDtypeStruct((NP, 1), _f32)],
    )(degp_t, h)

    acc = _agg1_call(g2, src_a, dst_a, w_a)

    zg = pl.pallas_call(
        _c_body,
        grid=(_GR,),
        in_specs=[pl.BlockSpec((1, _BS, 64), lambda i: (0, i, 0)),
                  pl.BlockSpec((1, _BS, 64), lambda i: (1, i, 0)),
                  pl.BlockSpec((1, _BS, 64), lambda i: (2, i, 0)),
                  pl.BlockSpec((1, _BS, 64), lambda i: (3, i, 0)),
                  pl.BlockSpec((_BS, 128), lambda i: (i, 0)),
                  pl.BlockSpec((_BS, 128), lambda i: (_GR + i, 0)),
                  pl.BlockSpec((_BS, 1), lambda i: (i, 0)),
                  pl.BlockSpec((D, 1), lambda i: (0, 0)),
                  pl.BlockSpec((1, D), lambda i: (0, 0))],
        out_specs=pl.BlockSpec((_BS, 1), lambda i: (i, 0)),
        out_shape=jax.ShapeDtypeStruct((NP, 1), _f32),
    )(acc, acc, acc, acc, g2, g2, dis, W2, b1.reshape(1, D))

    acc2 = _agg2_call(zg.reshape(NP), src_d, dst_d, w_d_flat)

    out = pl.pallas_call(
        _d_body,
        out_shape=jax.ShapeDtypeStruct((NP // 128, 128), _f32),
    )(acc2.reshape(NC, NP // 128, 128), zg.reshape(NP // 128, 128),
      dis.reshape(NP // 128, 128), b2.reshape(1, 1))

    return out.reshape(NP)[:N]
